# Initial kernel scaffold; baseline (speedup 1.0000x reference)
import jax
import jax.numpy as jnp
from jax.experimental import pallas as pl


def kernel(z, embedding, cluster_size):
    raise NotImplementedError("write your pallas kernel here")


# speedup vs baseline: 1.3966x; 1.3966x over previous
import jax
import jax.numpy as jnp
from jax.experimental import pallas as pl
from jax.experimental.pallas import tpu as pltpu

_TN = 256


def _vq_body(x_ref, embt_ref, emb_ref, cs_ref, qst_ref, idx_ref, stats_ref,
             counts_ref, ssq_ref, *, n_tokens, n_codes, nt_blocks):
    i = pl.program_id(0)
    x = x_ref[:]
    embt = embt_ref[:]

    xsq = jnp.sum(x * x, axis=1, keepdims=True)
    esq = jnp.sum(embt * embt, axis=0, keepdims=True)
    d2 = xsq - 2.0 * jnp.dot(x, embt, preferred_element_type=jnp.float32) + esq
    d2 = jnp.maximum(d2, 0.0)
    idx = jnp.argmin(d2, axis=1).astype(jnp.int32)

    iota = jax.lax.broadcasted_iota(jnp.int32, (x.shape[0], n_codes), 1)
    oh = (idx[:, None] == iota).astype(jnp.float32)
    q = jnp.dot(oh, emb_ref[:], preferred_element_type=jnp.float32)
    qst = x + (q - x)
    qst_ref[:] = qst
    idx_ref[:] = idx.reshape(1, 1, x.shape[0])

    cnt = jnp.sum(oh, axis=0, keepdims=True)
    diff = qst - x
    ssq = jnp.sum(diff * diff)

    @pl.when(i == 0)
    def _init():
        counts_ref[:] = cnt
        ssq_ref[0, 0] = ssq

    @pl.when(i > 0)
    def _accum():
        counts_ref[:] = counts_ref[:] + cnt
        ssq_ref[0, 0] = ssq_ref[0, 0] + ssq

    @pl.when(i == nt_blocks - 1)
    def _finish():
        counts = counts_ref[:]
        avg = counts * (1.0 / n_tokens)
        plogp = avg * jnp.log(avg + 1e-10)
        perp = jnp.exp(-jnp.sum(plogp))
        used = jnp.sum((cs_ref[:] > 1e-5).astype(jnp.float32)) * (1.0 / n_codes)
        loss = ssq_ref[0, 0] * (1.0 / (n_tokens * x.shape[1]))
        lane = jax.lax.broadcasted_iota(jnp.int32, (1, 8), 1)
        vec = jnp.where(lane == 0, loss,
                        jnp.where(lane == 1, perp,
                                  jnp.where(lane == 2, used, 0.0)))
        stats_ref[:] = vec


def kernel(z, embedding, cluster_size):
    B, C, D, H, W = z.shape
    K = embedding.shape[0]
    x = jnp.transpose(z, (0, 2, 3, 4, 1)).reshape(-1, C)
    N = x.shape[0]
    nt = N // _TN

    import functools
    body = functools.partial(_vq_body, n_tokens=N, n_codes=K, nt_blocks=nt)

    qst, idx3, stats = pl.pallas_call(
        body,
        grid=(nt,),
        in_specs=[
            pl.BlockSpec((_TN, C), lambda i: (i, 0)),
            pl.BlockSpec((C, K), lambda i: (0, 0)),
            pl.BlockSpec((K, C), lambda i: (0, 0)),
            pl.BlockSpec((1, K), lambda i: (0, 0)),
        ],
        out_specs=[
            pl.BlockSpec((_TN, C), lambda i: (i, 0)),
            pl.BlockSpec((1, 1, _TN), lambda i: (i, 0, 0)),
            pl.BlockSpec((1, 8), lambda i: (0, 0)),
        ],
        out_shape=[
            jax.ShapeDtypeStruct((N, C), jnp.float32),
            jax.ShapeDtypeStruct((nt, 1, _TN), jnp.int32),
            jax.ShapeDtypeStruct((1, 8), jnp.float32),
        ],
        scratch_shapes=[
            pltpu.VMEM((1, K), jnp.float32),
            pltpu.SMEM((1, 1), jnp.float32),
        ],
        compiler_params=pltpu.CompilerParams(
            dimension_semantics=("arbitrary",)),
    )(x, embedding.T, embedding, cluster_size.reshape(1, K))

    quantized_st = jnp.transpose(qst.reshape(B, D, H, W, C), (0, 4, 1, 2, 3))
    encoding_indices = idx3.reshape(B, D, H, W)
    return (quantized_st, stats[0, 0], encoding_indices, stats[0, 1],
            stats[0, 2])


# speedup vs baseline: 1.4757x; 1.0566x over previous
import jax
import jax.numpy as jnp
from jax.experimental import pallas as pl
from jax.experimental.pallas import tpu as pltpu

_TN = 512


def _vq_body(x_ref, embt_ref, emb_ref, cs_ref, qst_ref, idx_ref, stats_ref,
             counts_ref, ssq_ref, *, n_tokens, n_codes, nt_blocks):
    i = pl.program_id(0)
    x = x_ref[:]
    embt = embt_ref[:]

    xsq = jnp.sum(x * x, axis=1, keepdims=True)
    esq = jnp.sum(embt * embt, axis=0, keepdims=True)
    d2 = xsq - 2.0 * jnp.dot(x, embt, preferred_element_type=jnp.float32) + esq
    d2 = jnp.maximum(d2, 0.0)
    idx = jnp.argmin(d2, axis=1).astype(jnp.int32)

    iota = jax.lax.broadcasted_iota(jnp.int32, (x.shape[0], n_codes), 1)
    oh = (idx[:, None] == iota).astype(jnp.float32)
    q = jnp.dot(oh, emb_ref[:], preferred_element_type=jnp.float32)
    qst = x + (q - x)
    qst_ref[:] = qst
    idx_ref[:] = idx.reshape(1, 1, x.shape[0])

    cnt = jnp.sum(oh, axis=0, keepdims=True)
    diff = qst - x
    ssq = jnp.sum(diff * diff)

    @pl.when(i == 0)
    def _init():
        counts_ref[:] = cnt
        ssq_ref[0, 0] = ssq

    @pl.when(i > 0)
    def _accum():
        counts_ref[:] = counts_ref[:] + cnt
        ssq_ref[0, 0] = ssq_ref[0, 0] + ssq

    @pl.when(i == nt_blocks - 1)
    def _finish():
        counts = counts_ref[:]
        avg = counts * (1.0 / n_tokens)
        plogp = avg * jnp.log(avg + 1e-10)
        perp = jnp.exp(-jnp.sum(plogp))
        used = jnp.sum((cs_ref[:] > 1e-5).astype(jnp.float32)) * (1.0 / n_codes)
        loss = ssq_ref[0, 0] * (1.0 / (n_tokens * x.shape[1]))
        lane = jax.lax.broadcasted_iota(jnp.int32, (1, 8), 1)
        vec = jnp.where(lane == 0, loss,
                        jnp.where(lane == 1, perp,
                                  jnp.where(lane == 2, used, 0.0)))
        stats_ref[:] = vec


def kernel(z, embedding, cluster_size):
    B, C, D, H, W = z.shape
    K = embedding.shape[0]
    x = jnp.transpose(z, (0, 2, 3, 4, 1)).reshape(-1, C)
    N = x.shape[0]
    nt = N // _TN

    import functools
    body = functools.partial(_vq_body, n_tokens=N, n_codes=K, nt_blocks=nt)

    qst, idx3, stats = pl.pallas_call(
        body,
        grid=(nt,),
        in_specs=[
            pl.BlockSpec((_TN, C), lambda i: (i, 0)),
            pl.BlockSpec((C, K), lambda i: (0, 0)),
            pl.BlockSpec((K, C), lambda i: (0, 0)),
            pl.BlockSpec((1, K), lambda i: (0, 0)),
        ],
        out_specs=[
            pl.BlockSpec((_TN, C), lambda i: (i, 0)),
            pl.BlockSpec((1, 1, _TN), lambda i: (i, 0, 0)),
            pl.BlockSpec((1, 8), lambda i: (0, 0)),
        ],
        out_shape=[
            jax.ShapeDtypeStruct((N, C), jnp.float32),
            jax.ShapeDtypeStruct((nt, 1, _TN), jnp.int32),
            jax.ShapeDtypeStruct((1, 8), jnp.float32),
        ],
        scratch_shapes=[
            pltpu.VMEM((1, K), jnp.float32),
            pltpu.SMEM((1, 1), jnp.float32),
        ],
        compiler_params=pltpu.CompilerParams(
            dimension_semantics=("arbitrary",)),
    )(x, embedding.T, embedding, cluster_size.reshape(1, K))

    quantized_st = jnp.transpose(qst.reshape(B, D, H, W, C), (0, 4, 1, 2, 3))
    encoding_indices = idx3.reshape(B, D, H, W)
    return (quantized_st, stats[0, 0], encoding_indices, stats[0, 1],
            stats[0, 2])
